# baseline (speedup 1.0000x reference)
import functools

import jax
import jax.numpy as jnp
from jax import lax
from jax.experimental import pallas as pl
from jax.experimental.pallas import tpu as pltpu
from jax.experimental.pallas import tpu_sc as plsc

B = 4096
V = 100000
D = 64
WD = 16
DW = D + WD
PW = 128
NPOS = 10
NNEG = 20
NV = NPOS + 3 * NNEG
NVP = 72
BETA = 2.0

NC = 2
NS = 16
NW = NC * NS
BPW = B // NW
RPW = BPW * NVP
CHUNK = 128
CPB = 4
BLK = CHUNK * CPB
NBLK = RPW // BLK


def _sc_gather_body(v_hbm, u_hbm, user_hbm, vidx_hbm, uidx_hbm, useridx_hbm,
                    out_v, out_u, out_user,
                    vidx_v, uidx_v, useridx_v, vbuf, semg, semo):
    wid = lax.axis_index("s") * NC + lax.axis_index("c")
    vbase = wid * RPW
    bbase = wid * BPW

    pltpu.sync_copy(vidx_hbm.at[pl.ds(vbase, RPW)], vidx_v)
    pltpu.sync_copy(uidx_hbm.at[pl.ds(bbase, BPW)], uidx_v)
    pltpu.sync_copy(useridx_hbm.at[pl.ds(bbase, BPW)], useridx_v)

    cu = pltpu.async_copy(u_hbm.at[uidx_v], vbuf.at[pl.ds(0, BPW)], semg)
    cuser = pltpu.async_copy(user_hbm.at[useridx_v],
                             vbuf.at[pl.ds(BPW, BPW)], semg)
    cu.wait()
    cuser.wait()
    ou = pltpu.async_copy(vbuf.at[pl.ds(0, BPW)],
                          out_u.at[pl.ds(bbase, BPW)], semo)
    ouser = pltpu.async_copy(vbuf.at[pl.ds(BPW, BPW)],
                             out_user.at[pl.ds(bbase, BPW)], semo)
    ou.wait()
    ouser.wait()

    def blk(i, carry):
        rbase = i * BLK
        cps = []
        for c in range(CPB):
            idx_sl = vidx_v.at[pl.ds(rbase + c * CHUNK, CHUNK)]
            cps.append(pltpu.async_copy(
                v_hbm.at[idx_sl], vbuf.at[pl.ds(c * CHUNK, CHUNK)], semg))
        for cp in cps:
            cp.wait()
        oc = pltpu.async_copy(vbuf, out_v.at[pl.ds(vbase + rbase, BLK)], semo)
        oc.wait()
        return carry

    lax.fori_loop(0, NBLK, blk, 0)


@functools.cache
def _sc_gather():
    return pl.kernel(
        _sc_gather_body,
        out_type=[
            jax.ShapeDtypeStruct((B * NVP, PW), jnp.float32),
            jax.ShapeDtypeStruct((B, PW), jnp.float32),
            jax.ShapeDtypeStruct((B, PW), jnp.float32),
        ],
        mesh=plsc.VectorSubcoreMesh(core_axis_name="c", subcore_axis_name="s"),
        scratch_types=[
            pltpu.VMEM((RPW,), jnp.int32),
            pltpu.VMEM((BPW,), jnp.int32),
            pltpu.VMEM((BPW,), jnp.int32),
            pltpu.VMEM((BLK, PW), jnp.float32),
            pltpu.SemaphoreType.DMA,
            pltpu.SemaphoreType.DMA,
        ],
        compiler_params=pltpu.CompilerParams(use_tc_tiling_on_sc=True),
    )


def _logsig(x):
    return jnp.minimum(x, 0.0) - jnp.log1p(jnp.exp(-jnp.abs(x)))


def _tc_score_body(v_ref, u_ref, user_ref, wd_ref, week_ref, out_ref):
    bb = v_ref.shape[0]
    u = u_ref[...]
    wk = week_ref[...]
    wd = wd_ref[...]
    wrow = jnp.where(wd == 0, wk[0:1, :], wk[1:2, :])
    wpad = jnp.concatenate(
        [jnp.zeros((bb, D), jnp.float32), wrow,
         jnp.zeros((bb, PW - DW), jnp.float32)], axis=1)
    cat = u + wpad
    user = user_ref[...]
    t = jnp.sum(cat * user, axis=-1, keepdims=True)
    rows = v_ref[...]
    col3 = lax.broadcasted_iota(jnp.int32, (bb, NVP, 1), 1)
    m = jnp.where(col3 < NPOS + NNEG, cat[:, None, :], user[:, None, :])
    s = jnp.sum(rows * m, axis=-1)
    col = lax.broadcasted_iota(jnp.int32, (bb, NVP), 1)
    a = jnp.where(col < NPOS, s,
                  jnp.where(col < NPOS + NNEG, -s, t - s))
    w = jnp.where(col < NPOS + NNEG, 1.0,
                  jnp.where(col < NV, BETA, 0.0))
    part = jnp.sum(w * _logsig(a))

    @pl.when(pl.program_id(0) == 0)
    def _():
        out_ref[...] = jnp.zeros_like(out_ref)

    out_ref[...] = out_ref[...] - part


def _tc_score(rows3d, rows_u, rows_user, wd2d, week_emb, bb=256):
    nblk = B // bb
    return pl.pallas_call(
        _tc_score_body,
        grid=(nblk,),
        in_specs=[
            pl.BlockSpec((bb, NVP, PW), lambda i: (i, 0, 0)),
            pl.BlockSpec((bb, PW), lambda i: (i, 0)),
            pl.BlockSpec((bb, PW), lambda i: (i, 0)),
            pl.BlockSpec((bb, 1), lambda i: (i, 0)),
            pl.BlockSpec((2, WD), lambda i: (0, 0)),
        ],
        out_specs=pl.BlockSpec((1, 1), lambda i: (0, 0)),
        out_shape=jax.ShapeDtypeStruct((1, 1), jnp.float32),
    )(rows3d, rows_u, rows_user, wd2d, week_emb)


def kernel(pos_u, pos_v, neg_v, user, weekday, neg_ne, neg_nn,
           u_emb, v_emb, user_emb, week_emb):
    vidx = jnp.concatenate(
        [pos_v, neg_v, neg_ne, neg_nn,
         jnp.zeros((B, NVP - NV), pos_v.dtype)], axis=1)
    vidx = vidx.reshape(-1).astype(jnp.int32)
    v128 = jnp.pad(v_emb, ((0, 0), (0, PW - DW)))
    u128 = jnp.pad(u_emb, ((0, 0), (0, PW - D)))
    user128 = jnp.pad(user_emb, ((0, 0), (0, PW - DW)))
    rows_v, rows_u, rows_user = _sc_gather()(
        v128, u128, user128, vidx,
        pos_u.astype(jnp.int32), user.astype(jnp.int32))
    out = _tc_score(rows_v.reshape(B, NVP, PW), rows_u, rows_user,
                    weekday.reshape(B, 1).astype(jnp.int32), week_emb)
    return out[0, 0]


# speedup vs baseline: 1.1236x; 1.1236x over previous
import functools

import jax
import jax.numpy as jnp
from jax import lax
from jax.experimental import pallas as pl
from jax.experimental.pallas import tpu as pltpu
from jax.experimental.pallas import tpu_sc as plsc

B = 4096
V = 100000
D = 64
WD = 16
DW = D + WD
PW = 128
NPOS = 10
NNEG = 20
NV = NPOS + 3 * NNEG
NVP = 72
BETA = 2.0

NC = 2
NS = 16
NW = NC * NS
BPW = B // NW
RPW = BPW * NVP
CHUNK = 128
CPB = 4
BLK = CHUNK * CPB
NBLK = RPW // BLK


def _pad_body(x_ref, o_ref):
    blk = x_ref.shape[0]
    w = x_ref.shape[1]
    o_ref[...] = jnp.concatenate(
        [x_ref[...], jnp.zeros((blk, PW - w), jnp.float32)], axis=1)


def _pad128(x, rblk=4000):
    n, w = x.shape
    grid = pl.cdiv(n, rblk)
    return pl.pallas_call(
        _pad_body,
        grid=(grid,),
        in_specs=[pl.BlockSpec((rblk, w), lambda i: (i, 0))],
        out_specs=pl.BlockSpec((rblk, PW), lambda i: (i, 0)),
        out_shape=jax.ShapeDtypeStruct((n, PW), jnp.float32),
    )(x)


def _sc_gather_body(v_hbm, u_hbm, user_hbm, vidx_hbm, uidx_hbm, useridx_hbm,
                    out_v, out_u, out_user,
                    vidx_v, uidx_v, useridx_v, vbuf, semg, semo):
    wid = lax.axis_index("s") * NC + lax.axis_index("c")
    vbase = wid * RPW
    bbase = wid * BPW

    pltpu.sync_copy(vidx_hbm.at[pl.ds(vbase, RPW)], vidx_v)
    pltpu.sync_copy(uidx_hbm.at[pl.ds(bbase, BPW)], uidx_v)
    pltpu.sync_copy(useridx_hbm.at[pl.ds(bbase, BPW)], useridx_v)

    cu = pltpu.async_copy(u_hbm.at[uidx_v], vbuf.at[pl.ds(0, BPW)], semg)
    cuser = pltpu.async_copy(user_hbm.at[useridx_v],
                             vbuf.at[pl.ds(BPW, BPW)], semg)
    cu.wait()
    cuser.wait()
    ou = pltpu.async_copy(vbuf.at[pl.ds(0, BPW)],
                          out_u.at[pl.ds(bbase, BPW)], semo)
    ouser = pltpu.async_copy(vbuf.at[pl.ds(BPW, BPW)],
                             out_user.at[pl.ds(bbase, BPW)], semo)
    ou.wait()
    ouser.wait()

    def blk(i, carry):
        rbase = i * BLK
        cps = []
        for c in range(CPB):
            idx_sl = vidx_v.at[pl.ds(rbase + c * CHUNK, CHUNK)]
            cps.append(pltpu.async_copy(
                v_hbm.at[idx_sl], vbuf.at[pl.ds(c * CHUNK, CHUNK)], semg))
        for cp in cps:
            cp.wait()
        oc = pltpu.async_copy(vbuf, out_v.at[pl.ds(vbase + rbase, BLK)], semo)
        oc.wait()
        return carry

    lax.fori_loop(0, NBLK, blk, 0)


@functools.cache
def _sc_gather():
    return pl.kernel(
        _sc_gather_body,
        out_type=[
            jax.ShapeDtypeStruct((B * NVP, PW), jnp.float32),
            jax.ShapeDtypeStruct((B, PW), jnp.float32),
            jax.ShapeDtypeStruct((B, PW), jnp.float32),
        ],
        mesh=plsc.VectorSubcoreMesh(core_axis_name="c", subcore_axis_name="s"),
        scratch_types=[
            pltpu.VMEM((RPW,), jnp.int32),
            pltpu.VMEM((BPW,), jnp.int32),
            pltpu.VMEM((BPW,), jnp.int32),
            pltpu.VMEM((BLK, PW), jnp.float32),
            pltpu.SemaphoreType.DMA,
            pltpu.SemaphoreType.DMA,
        ],
        compiler_params=pltpu.CompilerParams(use_tc_tiling_on_sc=True),
    )


def _logsig(x):
    return jnp.minimum(x, 0.0) - jnp.log1p(jnp.exp(-jnp.abs(x)))


def _tc_score_body(v_ref, u_ref, user_ref, wd_ref, week_ref, out_ref):
    bb = v_ref.shape[0]
    u = u_ref[...]
    wk = week_ref[...]
    wd = wd_ref[...]
    wrow = jnp.where(wd == 0, wk[0:1, :], wk[1:2, :])
    wpad = jnp.concatenate(
        [jnp.zeros((bb, D), jnp.float32), wrow,
         jnp.zeros((bb, PW - DW), jnp.float32)], axis=1)
    cat = u + wpad
    user = user_ref[...]
    t = jnp.sum(cat * user, axis=-1, keepdims=True)
    rows = v_ref[...]
    col3 = lax.broadcasted_iota(jnp.int32, (bb, NVP, 1), 1)
    m = jnp.where(col3 < NPOS + NNEG, cat[:, None, :], user[:, None, :])
    s = jnp.sum(rows * m, axis=-1)
    col = lax.broadcasted_iota(jnp.int32, (bb, NVP), 1)
    a = jnp.where(col < NPOS, s,
                  jnp.where(col < NPOS + NNEG, -s, t - s))
    w = jnp.where(col < NPOS + NNEG, 1.0,
                  jnp.where(col < NV, BETA, 0.0))
    part = jnp.sum(w * _logsig(a))

    @pl.when(pl.program_id(0) == 0)
    def _():
        out_ref[...] = jnp.zeros_like(out_ref)

    out_ref[...] = out_ref[...] - part


def _tc_score(rows3d, rows_u, rows_user, wd2d, week_emb, bb=256):
    nblk = B // bb
    return pl.pallas_call(
        _tc_score_body,
        grid=(nblk,),
        in_specs=[
            pl.BlockSpec((bb, NVP, PW), lambda i: (i, 0, 0)),
            pl.BlockSpec((bb, PW), lambda i: (i, 0)),
            pl.BlockSpec((bb, PW), lambda i: (i, 0)),
            pl.BlockSpec((bb, 1), lambda i: (i, 0)),
            pl.BlockSpec((2, WD), lambda i: (0, 0)),
        ],
        out_specs=pl.BlockSpec((1, 1), lambda i: (0, 0)),
        out_shape=jax.ShapeDtypeStruct((1, 1), jnp.float32),
    )(rows3d, rows_u, rows_user, wd2d, week_emb)


def kernel(pos_u, pos_v, neg_v, user, weekday, neg_ne, neg_nn,
           u_emb, v_emb, user_emb, week_emb):
    vidx = jnp.concatenate(
        [pos_v, neg_v, neg_ne, neg_nn,
         jnp.zeros((B, NVP - NV), pos_v.dtype)], axis=1)
    vidx = vidx.reshape(-1).astype(jnp.int32)
    v128 = _pad128(v_emb)
    u128 = _pad128(u_emb)
    user128 = _pad128(user_emb)
    rows_v, rows_u, rows_user = _sc_gather()(
        v128, u128, user128, vidx,
        pos_u.astype(jnp.int32), user.astype(jnp.int32))
    out = _tc_score(rows_v.reshape(B, NVP, PW), rows_u, rows_user,
                    weekday.reshape(B, 1).astype(jnp.int32), week_emb)
    return out[0, 0]


# speedup vs baseline: 2.3777x; 2.1163x over previous
import functools

import jax
import jax.numpy as jnp
from jax import lax
from jax.experimental import pallas as pl
from jax.experimental.pallas import tpu as pltpu
from jax.experimental.pallas import tpu_sc as plsc

B = 4096
V = 100000
D = 64
WD = 16
DW = D + WD
PW = 128
NPOS = 10
NNEG = 20
NV = NPOS + 3 * NNEG
NVP = 72
BETA = 2.0

NC = 2
NS = 16
NW = NC * NS
BPW = B // NW
RPW = BPW * NVP
CHUNK = 128
CPB = 4
BLK = CHUNK * CPB
NBLK = RPW // BLK


def _pad_body(x_ref, o_ref):
    blk = x_ref.shape[0]
    w = x_ref.shape[1]
    o_ref[...] = jnp.concatenate(
        [x_ref[...], jnp.zeros((blk, PW - w), jnp.float32)], axis=1)


def _pad128(x, rblk=4000):
    n, w = x.shape
    grid = pl.cdiv(n, rblk)
    return pl.pallas_call(
        _pad_body,
        grid=(grid,),
        in_specs=[pl.BlockSpec((rblk, w), lambda i: (i, 0))],
        out_specs=pl.BlockSpec((rblk, PW), lambda i: (i, 0)),
        out_shape=jax.ShapeDtypeStruct((n, PW), jnp.float32),
    )(x)


def _sc_gather_body(v_hbm, u_hbm, user_hbm, vidx_hbm, uidx_hbm, useridx_hbm,
                    out_v, out_u, out_user,
                    vidx_v, uidx_v, useridx_v, vbuf, semg, semo):
    wid = lax.axis_index("s") * NC + lax.axis_index("c")
    vbase = wid * RPW
    bbase = wid * BPW

    pltpu.sync_copy(vidx_hbm.at[pl.ds(vbase, RPW)], vidx_v)
    pltpu.sync_copy(uidx_hbm.at[pl.ds(bbase, BPW)], uidx_v)
    pltpu.sync_copy(useridx_hbm.at[pl.ds(bbase, BPW)], useridx_v)

    cu = pltpu.async_copy(u_hbm.at[uidx_v], vbuf.at[pl.ds(0, BPW)], semg)
    cuser = pltpu.async_copy(user_hbm.at[useridx_v],
                             vbuf.at[pl.ds(BPW, BPW)], semg)
    cu.wait()
    cuser.wait()
    ou = pltpu.async_copy(vbuf.at[pl.ds(0, BPW)],
                          out_u.at[pl.ds(bbase, BPW)], semo)
    ouser = pltpu.async_copy(vbuf.at[pl.ds(BPW, BPW)],
                             out_user.at[pl.ds(bbase, BPW)], semo)
    ou.wait()
    ouser.wait()

    def blk(i, carry):
        rbase = i * BLK
        cps = []
        for c in range(CPB):
            idx_sl = vidx_v.at[pl.ds(rbase + c * CHUNK, CHUNK)]
            cps.append(pltpu.async_copy(
                v_hbm.at[idx_sl], vbuf.at[pl.ds(c * CHUNK, CHUNK)], semg))
        for cp in cps:
            cp.wait()
        oc = pltpu.async_copy(vbuf, out_v.at[pl.ds(vbase + rbase, BLK)], semo)
        oc.wait()
        return carry

    lax.fori_loop(0, NBLK, blk, 0)


@functools.cache
def _sc_gather():
    return pl.kernel(
        _sc_gather_body,
        out_type=[
            jax.ShapeDtypeStruct((B * NVP, PW), jnp.float32),
            jax.ShapeDtypeStruct((B, PW), jnp.float32),
            jax.ShapeDtypeStruct((B, PW), jnp.float32),
        ],
        mesh=plsc.VectorSubcoreMesh(core_axis_name="c", subcore_axis_name="s"),
        scratch_types=[
            pltpu.VMEM((RPW,), jnp.int32),
            pltpu.VMEM((BPW,), jnp.int32),
            pltpu.VMEM((BPW,), jnp.int32),
            pltpu.VMEM((BLK, PW), jnp.float32),
            pltpu.SemaphoreType.DMA,
            pltpu.SemaphoreType.DMA,
        ],
        compiler_params=pltpu.CompilerParams(use_tc_tiling_on_sc=True),
    )


def _logsig(x):
    return jnp.minimum(x, 0.0) - jnp.log1p(jnp.exp(-jnp.abs(x)))


def _tc_score_body(v_ref, u_ref, user_ref, wd_ref, week_ref, out_ref):
    bb = v_ref.shape[0]
    u = u_ref[...]
    wk = week_ref[...]
    wd = wd_ref[...]
    wrow = jnp.where(wd == 0, wk[0:1, :], wk[1:2, :])
    wpad = jnp.concatenate(
        [jnp.zeros((bb, D), jnp.float32), wrow,
         jnp.zeros((bb, PW - DW), jnp.float32)], axis=1)
    cat = u + wpad
    user = user_ref[...]
    t = jnp.sum(cat * user, axis=-1, keepdims=True)
    rows = v_ref[...]
    col3 = lax.broadcasted_iota(jnp.int32, (bb, NVP, 1), 1)
    m = jnp.where(col3 < NPOS + NNEG, cat[:, None, :], user[:, None, :])
    s = jnp.sum(rows * m, axis=-1)
    col = lax.broadcasted_iota(jnp.int32, (bb, NVP), 1)
    a = jnp.where(col < NPOS, s,
                  jnp.where(col < NPOS + NNEG, -s, t - s))
    w = jnp.where(col < NPOS + NNEG, 1.0,
                  jnp.where(col < NV, BETA, 0.0))
    part = jnp.sum(w * _logsig(a))

    @pl.when(pl.program_id(0) == 0)
    def _():
        out_ref[...] = jnp.zeros_like(out_ref)

    out_ref[...] = out_ref[...] - part


def _tc_score(rows3d, rows_u, rows_user, wd2d, week_emb, bb=256):
    nblk = B // bb
    return pl.pallas_call(
        _tc_score_body,
        grid=(nblk,),
        in_specs=[
            pl.BlockSpec((bb, NVP, PW), lambda i: (i, 0, 0)),
            pl.BlockSpec((bb, PW), lambda i: (i, 0)),
            pl.BlockSpec((bb, PW), lambda i: (i, 0)),
            pl.BlockSpec((bb, 1), lambda i: (i, 0)),
            pl.BlockSpec((2, WD), lambda i: (0, 0)),
        ],
        out_specs=pl.BlockSpec((1, 1), lambda i: (0, 0)),
        out_shape=jax.ShapeDtypeStruct((1, 1), jnp.float32),
    )(rows3d, rows_u, rows_user, wd2d, week_emb)


def kernel(pos_u, pos_v, neg_v, user, weekday, neg_ne, neg_nn,
           u_emb, v_emb, user_emb, week_emb):
    vidx = jnp.concatenate(
        [pos_v, neg_v, neg_ne, neg_nn, pos_v[:, :NVP - NV]], axis=1)
    vidx = vidx.reshape(-1).astype(jnp.int32)
    v128 = _pad128(v_emb)
    u128 = _pad128(u_emb)
    user128 = _pad128(user_emb)
    rows_v, rows_u, rows_user = _sc_gather()(
        v128, u128, user128, vidx,
        pos_u.astype(jnp.int32), user.astype(jnp.int32))
    out = _tc_score(rows_v.reshape(B, NVP, PW), rows_u, rows_user,
                    weekday.reshape(B, 1).astype(jnp.int32), week_emb)
    return out[0, 0]


# speedup vs baseline: 2.3862x; 1.0036x over previous
import functools

import jax
import jax.numpy as jnp
from jax import lax
from jax.experimental import pallas as pl
from jax.experimental.pallas import tpu as pltpu
from jax.experimental.pallas import tpu_sc as plsc

B = 4096
V = 100000
D = 64
WD = 16
DW = D + WD
PW = 128
NPOS = 10
NNEG = 20
NV = NPOS + 3 * NNEG
NVP = 72
BETA = 2.0

NC = 2
NS = 16
NW = NC * NS
BPW = B // NW
RPW = BPW * NVP
CHUNK = 128
CPB = 4
BLK = CHUNK * CPB
NBLK = RPW // BLK


def _pad_body(x_ref, o_ref):
    blk = x_ref.shape[0]
    w = x_ref.shape[1]
    o_ref[...] = jnp.concatenate(
        [x_ref[...], jnp.zeros((blk, PW - w), jnp.float32)], axis=1)


def _pad128(x, rblk=4000):
    n, w = x.shape
    grid = pl.cdiv(n, rblk)
    return pl.pallas_call(
        _pad_body,
        grid=(grid,),
        in_specs=[pl.BlockSpec((rblk, w), lambda i: (i, 0))],
        out_specs=pl.BlockSpec((rblk, PW), lambda i: (i, 0)),
        out_shape=jax.ShapeDtypeStruct((n, PW), jnp.float32),
    )(x)


def _sc_gather_v_body(v_hbm, vidx_hbm, out_v, vidx_v, vbuf, semg, semo):
    wid = lax.axis_index("s") * NC + lax.axis_index("c")
    vbase = wid * RPW
    pltpu.sync_copy(vidx_hbm.at[pl.ds(vbase, RPW)], vidx_v)

    def blk(i, carry):
        rbase = i * BLK
        cps = []
        for c in range(CPB):
            idx_sl = vidx_v.at[pl.ds(rbase + c * CHUNK, CHUNK)]
            cps.append(pltpu.async_copy(
                v_hbm.at[idx_sl], vbuf.at[pl.ds(c * CHUNK, CHUNK)], semg))
        for cp in cps:
            cp.wait()
        oc = pltpu.async_copy(vbuf, out_v.at[pl.ds(vbase + rbase, BLK)], semo)
        oc.wait()
        return carry

    lax.fori_loop(0, NBLK, blk, 0)


@functools.cache
def _sc_gather_v():
    return pl.kernel(
        _sc_gather_v_body,
        out_type=jax.ShapeDtypeStruct((B * NVP, PW), jnp.float32),
        mesh=plsc.VectorSubcoreMesh(core_axis_name="c", subcore_axis_name="s"),
        scratch_types=[
            pltpu.VMEM((RPW,), jnp.int32),
            pltpu.VMEM((BLK, PW), jnp.float32),
            pltpu.SemaphoreType.DMA,
            pltpu.SemaphoreType.DMA,
        ],
        compiler_params=pltpu.CompilerParams(use_tc_tiling_on_sc=True),
    )


def _sc_gather_uu_body(u_hbm, user_hbm, uidx_hbm, useridx_hbm,
                       out_u, out_user, uidx_v, useridx_v, buf, semg, semo):
    wid = lax.axis_index("s") * NC + lax.axis_index("c")
    bbase = wid * BPW
    pltpu.sync_copy(uidx_hbm.at[pl.ds(bbase, BPW)], uidx_v)
    pltpu.sync_copy(useridx_hbm.at[pl.ds(bbase, BPW)], useridx_v)
    cu = pltpu.async_copy(u_hbm.at[uidx_v], buf.at[pl.ds(0, BPW)], semg)
    cuser = pltpu.async_copy(user_hbm.at[useridx_v],
                             buf.at[pl.ds(BPW, BPW)], semg)
    cu.wait()
    cuser.wait()
    ou = pltpu.async_copy(buf.at[pl.ds(0, BPW)],
                          out_u.at[pl.ds(bbase, BPW)], semo)
    ouser = pltpu.async_copy(buf.at[pl.ds(BPW, BPW)],
                             out_user.at[pl.ds(bbase, BPW)], semo)
    ou.wait()
    ouser.wait()


@functools.cache
def _sc_gather_uu():
    return pl.kernel(
        _sc_gather_uu_body,
        out_type=[
            jax.ShapeDtypeStruct((B, PW), jnp.float32),
            jax.ShapeDtypeStruct((B, PW), jnp.float32),
        ],
        mesh=plsc.VectorSubcoreMesh(core_axis_name="c", subcore_axis_name="s"),
        scratch_types=[
            pltpu.VMEM((BPW,), jnp.int32),
            pltpu.VMEM((BPW,), jnp.int32),
            pltpu.VMEM((2 * BPW, PW), jnp.float32),
            pltpu.SemaphoreType.DMA,
            pltpu.SemaphoreType.DMA,
        ],
        compiler_params=pltpu.CompilerParams(use_tc_tiling_on_sc=True),
    )


def _logsig(x):
    return jnp.minimum(x, 0.0) - jnp.log1p(jnp.exp(-jnp.abs(x)))


def _tc_score_body(v_ref, u_ref, user_ref, wd_ref, week_ref, out_ref):
    bb = v_ref.shape[0]
    u = u_ref[...]
    wk = week_ref[...]
    wd = wd_ref[...]
    wrow = jnp.where(wd == 0, wk[0:1, :], wk[1:2, :])
    wpad = jnp.concatenate(
        [jnp.zeros((bb, D), jnp.float32), wrow,
         jnp.zeros((bb, PW - DW), jnp.float32)], axis=1)
    cat = u + wpad
    user = user_ref[...]
    t = jnp.sum(cat * user, axis=-1, keepdims=True)
    rows = v_ref[...]
    col3 = lax.broadcasted_iota(jnp.int32, (bb, NVP, 1), 1)
    m = jnp.where(col3 < NPOS + NNEG, cat[:, None, :], user[:, None, :])
    s = jnp.sum(rows * m, axis=-1)
    col = lax.broadcasted_iota(jnp.int32, (bb, NVP), 1)
    a = jnp.where(col < NPOS, s,
                  jnp.where(col < NPOS + NNEG, -s, t - s))
    w = jnp.where(col < NPOS + NNEG, 1.0,
                  jnp.where(col < NV, BETA, 0.0))
    part = jnp.sum(w * _logsig(a))

    @pl.when(pl.program_id(0) == 0)
    def _():
        out_ref[...] = jnp.zeros_like(out_ref)

    out_ref[...] = out_ref[...] - part


def _tc_score(rows3d, rows_u, rows_user, wd2d, week_emb, bb=256):
    nblk = B // bb
    return pl.pallas_call(
        _tc_score_body,
        grid=(nblk,),
        in_specs=[
            pl.BlockSpec((bb, NVP, PW), lambda i: (i, 0, 0)),
            pl.BlockSpec((bb, PW), lambda i: (i, 0)),
            pl.BlockSpec((bb, PW), lambda i: (i, 0)),
            pl.BlockSpec((bb, 1), lambda i: (i, 0)),
            pl.BlockSpec((2, WD), lambda i: (0, 0)),
        ],
        out_specs=pl.BlockSpec((1, 1), lambda i: (0, 0)),
        out_shape=jax.ShapeDtypeStruct((1, 1), jnp.float32),
    )(rows3d, rows_u, rows_user, wd2d, week_emb)


def kernel(pos_u, pos_v, neg_v, user, weekday, neg_ne, neg_nn,
           u_emb, v_emb, user_emb, week_emb):
    vidx = jnp.concatenate(
        [pos_v, neg_v, neg_ne, neg_nn, pos_v[:, :NVP - NV]], axis=1)
    vidx = vidx.reshape(-1).astype(jnp.int32)
    v128 = _pad128(v_emb)
    rows_v = _sc_gather_v()(v128, vidx)
    u128 = _pad128(u_emb)
    user128 = _pad128(user_emb)
    rows_u, rows_user = _sc_gather_uu()(
        u128, user128, pos_u.astype(jnp.int32), user.astype(jnp.int32))
    out = _tc_score(rows_v.reshape(B, NVP, PW), rows_u, rows_user,
                    weekday.reshape(B, 1).astype(jnp.int32), week_emb)
    return out[0, 0]


# speedup vs baseline: 2.4340x; 1.0200x over previous
import functools

import jax
import jax.numpy as jnp
from jax import lax
from jax.experimental import pallas as pl
from jax.experimental.pallas import tpu as pltpu
from jax.experimental.pallas import tpu_sc as plsc

B = 4096
V = 100000
D = 64
WD = 16
DW = D + WD
PW = 128
NPOS = 10
NNEG = 20
NV = NPOS + 3 * NNEG
NVP = 72
BETA = 2.0

NC = 2
NS = 16
NW = NC * NS
BPW = B // NW
RPW = BPW * NVP
CHUNK = 128
CPB = 3
BLK = CHUNK * CPB
NBLK = RPW // BLK


def _pad_body(x_ref, o_ref):
    blk = x_ref.shape[0]
    w = x_ref.shape[1]
    o_ref[...] = jnp.concatenate(
        [x_ref[...], jnp.zeros((blk, PW - w), jnp.float32)], axis=1)


def _pad128(x, rblk=4000):
    n, w = x.shape
    grid = pl.cdiv(n, rblk)
    return pl.pallas_call(
        _pad_body,
        grid=(grid,),
        in_specs=[pl.BlockSpec((rblk, w), lambda i: (i, 0))],
        out_specs=pl.BlockSpec((rblk, PW), lambda i: (i, 0)),
        out_shape=jax.ShapeDtypeStruct((n, PW), jnp.float32),
    )(x)


def _sc_gather_v_body(v_hbm, vidx_hbm, out_v,
                      vidx_v, vbufa, vbufb, semg, semoa, semob):
    wid = lax.axis_index("s") * NC + lax.axis_index("c")
    vbase = wid * RPW
    pltpu.sync_copy(vidx_hbm.at[pl.ds(vbase, RPW)], vidx_v)

    def gather_blk(j, buf):
        rbase = j * BLK
        cps = []
        for c in range(CPB):
            idx_sl = vidx_v.at[pl.ds(rbase + c * CHUNK, CHUNK)]
            cps.append(pltpu.async_copy(
                v_hbm.at[idx_sl], buf.at[pl.ds(c * CHUNK, CHUNK)], semg))
        for cp in cps:
            cp.wait()

    def drain(buf, sem):
        pltpu.make_async_copy(out_v.at[pl.ds(vbase, BLK)], buf, sem).wait()

    def blk2(i, carry):
        @pl.when(i > 0)
        def _():
            drain(vbufa, semoa)
        gather_blk(2 * i, vbufa)

        @pl.when(i > 0)
        def _():
            drain(vbufb, semob)
        pltpu.async_copy(vbufa, out_v.at[pl.ds(vbase + 2 * i * BLK, BLK)],
                         semoa)
        gather_blk(2 * i + 1, vbufb)
        pltpu.async_copy(vbufb, out_v.at[pl.ds(vbase + (2 * i + 1) * BLK, BLK)],
                         semob)
        return carry

    lax.fori_loop(0, NBLK // 2, blk2, 0)
    drain(vbufa, semoa)
    drain(vbufb, semob)


@functools.cache
def _sc_gather_v():
    return pl.kernel(
        _sc_gather_v_body,
        out_type=jax.ShapeDtypeStruct((B * NVP, PW), jnp.float32),
        mesh=plsc.VectorSubcoreMesh(core_axis_name="c", subcore_axis_name="s"),
        scratch_types=[
            pltpu.VMEM((RPW,), jnp.int32),
            pltpu.VMEM((BLK, PW), jnp.float32),
            pltpu.VMEM((BLK, PW), jnp.float32),
            pltpu.SemaphoreType.DMA,
            pltpu.SemaphoreType.DMA,
            pltpu.SemaphoreType.DMA,
        ],
        compiler_params=pltpu.CompilerParams(use_tc_tiling_on_sc=True),
    )


def _sc_gather_uu_body(u_hbm, user_hbm, uidx_hbm, useridx_hbm,
                       out_u, out_user, uidx_v, useridx_v, buf, semg, semo):
    wid = lax.axis_index("s") * NC + lax.axis_index("c")
    bbase = wid * BPW
    pltpu.sync_copy(uidx_hbm.at[pl.ds(bbase, BPW)], uidx_v)
    pltpu.sync_copy(useridx_hbm.at[pl.ds(bbase, BPW)], useridx_v)
    cu = pltpu.async_copy(u_hbm.at[uidx_v], buf.at[pl.ds(0, BPW)], semg)
    cuser = pltpu.async_copy(user_hbm.at[useridx_v],
                             buf.at[pl.ds(BPW, BPW)], semg)
    cu.wait()
    cuser.wait()
    ou = pltpu.async_copy(buf.at[pl.ds(0, BPW)],
                          out_u.at[pl.ds(bbase, BPW)], semo)
    ouser = pltpu.async_copy(buf.at[pl.ds(BPW, BPW)],
                             out_user.at[pl.ds(bbase, BPW)], semo)
    ou.wait()
    ouser.wait()


@functools.cache
def _sc_gather_uu():
    return pl.kernel(
        _sc_gather_uu_body,
        out_type=[
            jax.ShapeDtypeStruct((B, PW), jnp.float32),
            jax.ShapeDtypeStruct((B, PW), jnp.float32),
        ],
        mesh=plsc.VectorSubcoreMesh(core_axis_name="c", subcore_axis_name="s"),
        scratch_types=[
            pltpu.VMEM((BPW,), jnp.int32),
            pltpu.VMEM((BPW,), jnp.int32),
            pltpu.VMEM((2 * BPW, PW), jnp.float32),
            pltpu.SemaphoreType.DMA,
            pltpu.SemaphoreType.DMA,
        ],
        compiler_params=pltpu.CompilerParams(use_tc_tiling_on_sc=True),
    )


def _logsig(x):
    return jnp.minimum(x, 0.0) - jnp.log1p(jnp.exp(-jnp.abs(x)))


def _tc_score_body(v_ref, u_ref, user_ref, wd_ref, week_ref, out_ref):
    bb = v_ref.shape[0]
    u = u_ref[...]
    wk = week_ref[...]
    wd = wd_ref[...]
    wrow = jnp.where(wd == 0, wk[0:1, :], wk[1:2, :])
    wpad = jnp.concatenate(
        [jnp.zeros((bb, D), jnp.float32), wrow,
         jnp.zeros((bb, PW - DW), jnp.float32)], axis=1)
    cat = u + wpad
    user = user_ref[...]
    t = jnp.sum(cat * user, axis=-1, keepdims=True)
    rows = v_ref[...]
    col3 = lax.broadcasted_iota(jnp.int32, (bb, NVP, 1), 1)
    m = jnp.where(col3 < NPOS + NNEG, cat[:, None, :], user[:, None, :])
    s = jnp.sum(rows * m, axis=-1)
    col = lax.broadcasted_iota(jnp.int32, (bb, NVP), 1)
    a = jnp.where(col < NPOS, s,
                  jnp.where(col < NPOS + NNEG, -s, t - s))
    w = jnp.where(col < NPOS + NNEG, 1.0,
                  jnp.where(col < NV, BETA, 0.0))
    part = jnp.sum(w * _logsig(a))

    @pl.when(pl.program_id(0) == 0)
    def _():
        out_ref[...] = jnp.zeros_like(out_ref)

    out_ref[...] = out_ref[...] - part


def _tc_score(rows3d, rows_u, rows_user, wd2d, week_emb, bb=256):
    nblk = B // bb
    return pl.pallas_call(
        _tc_score_body,
        grid=(nblk,),
        in_specs=[
            pl.BlockSpec((bb, NVP, PW), lambda i: (i, 0, 0)),
            pl.BlockSpec((bb, PW), lambda i: (i, 0)),
            pl.BlockSpec((bb, PW), lambda i: (i, 0)),
            pl.BlockSpec((bb, 1), lambda i: (i, 0)),
            pl.BlockSpec((2, WD), lambda i: (0, 0)),
        ],
        out_specs=pl.BlockSpec((1, 1), lambda i: (0, 0)),
        out_shape=jax.ShapeDtypeStruct((1, 1), jnp.float32),
    )(rows3d, rows_u, rows_user, wd2d, week_emb)


def kernel(pos_u, pos_v, neg_v, user, weekday, neg_ne, neg_nn,
           u_emb, v_emb, user_emb, week_emb):
    vidx = jnp.concatenate(
        [pos_v, neg_v, neg_ne, neg_nn, pos_v[:, :NVP - NV]], axis=1)
    vidx = vidx.reshape(-1).astype(jnp.int32)
    v128 = _pad128(v_emb)
    rows_v = _sc_gather_v()(v128, vidx)
    u128 = _pad128(u_emb)
    user128 = _pad128(user_emb)
    rows_u, rows_user = _sc_gather_uu()(
        u128, user128, pos_u.astype(jnp.int32), user.astype(jnp.int32))
    out = _tc_score(rows_v.reshape(B, NVP, PW), rows_u, rows_user,
                    weekday.reshape(B, 1).astype(jnp.int32), week_emb)
    return out[0, 0]
